# Initial kernel scaffold; baseline (speedup 1.0000x reference)
import jax
import jax.numpy as jnp
from jax.experimental import pallas as pl


def kernel(normalized_features, logits, knn_logits, labels, W1, b1, W2, b2):
    raise NotImplementedError("write your pallas kernel here")


# speedup vs baseline: 1.4088x; 1.4088x over previous
import functools

import jax
import jax.numpy as jnp
from jax.experimental import pallas as pl
from jax.experimental.pallas import tpu as pltpu

B = 1024
V = 100000
FEAT_DIM = 512
TOP_K = 30
NORM_DIM = 16

SEG_W = 200
N_SEG = V // SEG_W
RB = 8
NEG = float("-inf")
BIGI = 2 ** 30


def _k1_body(log_ref, knn_ref, lab_ref, feat_ref, w1_ref, b1_ref,
             w2f_ref, b2_ref, ids_ref, featdot_ref, target_ref):
    x = log_ref[...]
    y = knn_ref[...]
    segl = jnp.max(x, axis=2)
    segk = jnp.max(y, axis=2)

    iota = jax.lax.broadcasted_iota(jnp.int32, (RB, N_SEG), 1)
    s = segl
    ids = []
    for _ in range(TOP_K):
        m = jnp.max(s, axis=1, keepdims=True)
        pos = jnp.min(jnp.where(s == m, iota, BIGI), axis=1, keepdims=True)
        ids.append(pos)
        s = jnp.where(iota == pos, NEG, s)
    ids_ref[...] = jnp.concatenate(ids, axis=1)

    iota200 = jax.lax.broadcasted_iota(jnp.int32, (1, 1, SEG_W), 2)

    def row_argmax(ref, seg):
        m0 = jnp.max(seg, axis=1, keepdims=True)
        sstar = jnp.min(jnp.where(seg == m0, iota, BIGI), axis=1,
                        keepdims=True)
        outs = []
        for r in range(RB):
            sv = sstar[r, 0]
            chunk = ref[r, pl.ds(sv, 1), :]
            eq = chunk.reshape(1, 1, SEG_W) == m0[r, 0]
            p = jnp.min(jnp.where(eq, iota200, BIGI))
            outs.append((sv * SEG_W + p).reshape(1, 1))
        return jnp.concatenate(outs, axis=0)

    amaxl = row_argmax(log_ref, segl)
    amaxk = row_argmax(knn_ref, segk)
    lab = lab_ref[...]
    target_ref[...] = jnp.logical_and(amaxl != lab, amaxk == lab)

    h = jax.lax.dot_general(feat_ref[...], w1_ref[...],
                            (((1,), (1,)), ((), ())),
                            preferred_element_type=jnp.float32)
    h = h + b1_ref[...]
    h = jnp.where(h >= 0, h, 0.1 * h)
    featdot_ref[...] = (
        jnp.sum(h * w2f_ref[...], axis=1, keepdims=True) + b2_ref[0, 0])


def _k3_body(ids_ref, *refs):
    ins = refs[:TOP_K]
    out_ref = refs[TOP_K]
    for t in range(TOP_K):
        out_ref[0, 0, t * SEG_W:(t + 1) * SEG_W] = ins[t][0, 0, 0, :]


def _k4_body(cand_ref, featdot_ref, w2t_ref, out_ref):
    c = cand_ref[:, 0, :]
    n = TOP_K * SEG_W
    iota = jax.lax.broadcasted_iota(jnp.int32, (RB, n), 1)
    acc = featdot_ref[...]
    for t in range(TOP_K):
        m = jnp.max(c, axis=1, keepdims=True)
        acc = acc + m * w2t_ref[0, t]
        pos = jnp.min(jnp.where(c == m, iota, BIGI), axis=1, keepdims=True)
        c = jnp.where(iota == pos, NEG, c)
    out_ref[...] = acc


def kernel(normalized_features, logits, knn_logits, labels, W1, b1, W2, b2):
    nb = B // RB
    logits3 = logits.reshape(B, N_SEG, SEG_W)
    knn3 = knn_logits.reshape(B, N_SEG, SEG_W)
    labels2 = labels.astype(jnp.int32).reshape(B, 1)
    w2t = W2[:, :TOP_K]
    w2f = W2[:, TOP_K:]

    seg_ids, featdot, target = pl.pallas_call(
        _k1_body,
        grid=(nb,),
        in_specs=[
            pl.BlockSpec((RB, N_SEG, SEG_W), lambda i: (i, 0, 0)),
            pl.BlockSpec((RB, N_SEG, SEG_W), lambda i: (i, 0, 0)),
            pl.BlockSpec((RB, 1), lambda i: (i, 0)),
            pl.BlockSpec((RB, FEAT_DIM), lambda i: (i, 0)),
            pl.BlockSpec((NORM_DIM, FEAT_DIM), lambda i: (0, 0)),
            pl.BlockSpec((1, NORM_DIM), lambda i: (0, 0)),
            pl.BlockSpec((1, NORM_DIM), lambda i: (0, 0)),
            pl.BlockSpec((1, 1), lambda i: (0, 0)),
        ],
        out_specs=[
            pl.BlockSpec((RB, TOP_K), lambda i: (i, 0)),
            pl.BlockSpec((RB, 1), lambda i: (i, 0)),
            pl.BlockSpec((RB, 1), lambda i: (i, 0)),
        ],
        out_shape=[
            jax.ShapeDtypeStruct((B, TOP_K), jnp.int32),
            jax.ShapeDtypeStruct((B, 1), jnp.float32),
            jax.ShapeDtypeStruct((B, 1), jnp.bool_),
        ],
    )(logits3, knn3, labels2, normalized_features,
      W1, b1.reshape(1, NORM_DIM), w2f, b2.reshape(1, 1))

    logits4 = logits.reshape(B, N_SEG, 1, SEG_W)

    def chunk_map(i, ids, t):
        return (i, ids[i, t], 0, 0)

    gathered = pl.pallas_call(
        _k3_body,
        grid_spec=pltpu.PrefetchScalarGridSpec(
            num_scalar_prefetch=1,
            grid=(B,),
            in_specs=[
                pl.BlockSpec((1, 1, 1, SEG_W),
                             functools.partial(chunk_map, t=t))
                for t in range(TOP_K)
            ],
            out_specs=pl.BlockSpec((1, 1, TOP_K * SEG_W),
                                   lambda i, ids: (i, 0, 0)),
        ),
        out_shape=jax.ShapeDtypeStruct((B, 1, TOP_K * SEG_W), jnp.float32),
    )(seg_ids, *([logits4] * TOP_K))

    out = pl.pallas_call(
        _k4_body,
        grid=(nb,),
        in_specs=[
            pl.BlockSpec((RB, 1, TOP_K * SEG_W), lambda i: (i, 0, 0)),
            pl.BlockSpec((RB, 1), lambda i: (i, 0)),
            pl.BlockSpec((1, TOP_K), lambda i: (0, 0)),
        ],
        out_specs=pl.BlockSpec((RB, 1), lambda i: (i, 0)),
        out_shape=jax.ShapeDtypeStruct((B, 1), jnp.float32),
    )(gathered, featdot, w2t)

    return out.reshape(B), target.reshape(B)


# speedup vs baseline: 2.0291x; 1.4403x over previous
import functools

import jax
import jax.numpy as jnp
from jax import lax
from jax.experimental import pallas as pl
from jax.experimental.pallas import tpu as pltpu
from jax.experimental.pallas import tpu_sc as plsc

B = 1024
V = 100000
FEAT_DIM = 512
TOP_K = 30
NORM_DIM = 16

SEG_W = 200
N_SEG = V // SEG_W
RB = 8
NEG = float("-inf")
BIGI = 2 ** 30


def _k1_body(log_ref, knn_ref, lab_ref, feat_ref, w1_ref, b1_ref,
             w2f_ref, b2_ref, cand_ref, featdot_ref, target_ref):
    x = log_ref[...]
    y = knn_ref[...]
    segl = jnp.max(x, axis=2)
    segk = jnp.max(y, axis=2)

    iota = jax.lax.broadcasted_iota(jnp.int32, (RB, N_SEG), 1)
    s = segl
    for t in range(TOP_K):
        m = jnp.max(s, axis=1, keepdims=True)
        pos = jnp.min(jnp.where(s == m, iota, BIGI), axis=1, keepdims=True)
        s = jnp.where(iota == pos, NEG, s)
        for r in range(RB):
            sv = pos[r, 0]
            cand_ref[pl.ds(r, 1), pl.ds(t, 1), :] = (
                log_ref[pl.ds(r, 1), pl.ds(sv, 1), :])

    iota200 = jax.lax.broadcasted_iota(jnp.int32, (1, 1, SEG_W), 2)

    def row_argmax(ref, seg):
        m0 = jnp.max(seg, axis=1, keepdims=True)
        sstar = jnp.min(jnp.where(seg == m0, iota, BIGI), axis=1,
                        keepdims=True)
        outs = []
        for r in range(RB):
            sv = sstar[r, 0]
            chunk = ref[r, pl.ds(sv, 1), :]
            eq = chunk.reshape(1, 1, SEG_W) == m0[r, 0]
            p = jnp.min(jnp.where(eq, iota200, BIGI))
            outs.append((sv * SEG_W + p).reshape(1, 1))
        return jnp.concatenate(outs, axis=0)

    amaxl = row_argmax(log_ref, segl)
    amaxk = row_argmax(knn_ref, segk)
    lab = lab_ref[...]
    target_ref[...] = jnp.logical_and(amaxl != lab, amaxk == lab)

    h = jax.lax.dot_general(feat_ref[...], w1_ref[...],
                            (((1,), (1,)), ((), ())),
                            preferred_element_type=jnp.float32)
    h = h + b1_ref[...]
    h = jnp.where(h >= 0, h, 0.1 * h)
    featdot_ref[...] = (
        jnp.sum(h * w2f_ref[...], axis=1, keepdims=True) + b2_ref[0, 0])


def _k4_body(cand_ref, featdot_ref, w2t_ref, out_ref):
    c = cand_ref[...]
    chunk_iota = jax.lax.broadcasted_iota(jnp.int32, (RB, TOP_K, SEG_W), 1)
    pos_iota = chunk_iota * SEG_W + jax.lax.broadcasted_iota(
        jnp.int32, (RB, TOP_K, SEG_W), 2)
    acc = featdot_ref[...]
    for t in range(TOP_K):
        m = jnp.max(c, axis=(1, 2), keepdims=True)
        acc = acc + m[:, :, 0] * w2t_ref[0, t]
        pos = jnp.min(jnp.where(c == m, pos_iota, BIGI), axis=(1, 2),
                      keepdims=True)
        c = jnp.where(pos_iota == pos, NEG, c)
    out_ref[...] = acc


def kernel(normalized_features, logits, knn_logits, labels, W1, b1, W2, b2):
    nb = B // RB
    logits3 = logits.reshape(B, N_SEG, SEG_W)
    knn3 = knn_logits.reshape(B, N_SEG, SEG_W)
    labels2 = labels.astype(jnp.int32).reshape(B, 1)
    w2t = W2[:, :TOP_K]
    w2f = W2[:, TOP_K:]

    cands, featdot, target = pl.pallas_call(
        _k1_body,
        grid=(nb,),
        in_specs=[
            pl.BlockSpec((RB, N_SEG, SEG_W), lambda i: (i, 0, 0)),
            pl.BlockSpec((RB, N_SEG, SEG_W), lambda i: (i, 0, 0)),
            pl.BlockSpec((RB, 1), lambda i: (i, 0)),
            pl.BlockSpec((RB, FEAT_DIM), lambda i: (i, 0)),
            pl.BlockSpec((NORM_DIM, FEAT_DIM), lambda i: (0, 0)),
            pl.BlockSpec((1, NORM_DIM), lambda i: (0, 0)),
            pl.BlockSpec((1, NORM_DIM), lambda i: (0, 0)),
            pl.BlockSpec((1, 1), lambda i: (0, 0)),
        ],
        out_specs=[
            pl.BlockSpec((RB, TOP_K, SEG_W), lambda i: (i, 0, 0)),
            pl.BlockSpec((RB, 1), lambda i: (i, 0)),
            pl.BlockSpec((RB, 1), lambda i: (i, 0)),
        ],
        out_shape=[
            jax.ShapeDtypeStruct((B, TOP_K, SEG_W), jnp.float32),
            jax.ShapeDtypeStruct((B, 1), jnp.float32),
            jax.ShapeDtypeStruct((B, 1), jnp.bool_),
        ],
    )(logits3, knn3, labels2, normalized_features,
      W1, b1.reshape(1, NORM_DIM), w2f, b2.reshape(1, 1))

    out = pl.pallas_call(
        _k4_body,
        grid=(nb,),
        in_specs=[
            pl.BlockSpec((RB, TOP_K, SEG_W), lambda i: (i, 0, 0)),
            pl.BlockSpec((RB, 1), lambda i: (i, 0)),
            pl.BlockSpec((1, TOP_K), lambda i: (0, 0)),
        ],
        out_specs=pl.BlockSpec((RB, 1), lambda i: (i, 0)),
        out_shape=jax.ShapeDtypeStruct((B, 1), jnp.float32),
    )(cands, featdot, w2t)

    return out.reshape(B), target.reshape(B)


# speedup vs baseline: 2.0827x; 1.0264x over previous
import functools

import jax
import jax.numpy as jnp
from jax import lax
from jax.experimental import pallas as pl
from jax.experimental.pallas import tpu as pltpu
from jax.experimental.pallas import tpu_sc as plsc

B = 1024
V = 100000
FEAT_DIM = 512
TOP_K = 30
NORM_DIM = 16

SEG_W = 125
N_SEG = V // SEG_W
RB = 8
NEG = float("-inf")
BIGI = 2 ** 30


def _k1_body(log_ref, knn_ref, lab_ref, feat_ref, w1_ref, b1_ref,
             w2f_ref, b2_ref, cand_ref, featdot_ref, target_ref):
    x = log_ref[...]
    segl = jnp.max(x, axis=2)

    iota = jax.lax.broadcasted_iota(jnp.int32, (RB, N_SEG), 1)
    s = segl
    for t in range(TOP_K):
        m = jnp.max(s, axis=1, keepdims=True)
        pos = jnp.min(jnp.where(s == m, iota, BIGI), axis=1, keepdims=True)
        s = jnp.where(iota == pos, NEG, s)
        for r in range(RB):
            sv = pos[r, 0]
            cand_ref[pl.ds(r, 1), pl.ds(t, 1), :] = (
                log_ref[pl.ds(r, 1), pl.ds(sv, 1), :])

    iota200 = jax.lax.broadcasted_iota(jnp.int32, (1, 1, SEG_W), 2)

    def row_argmax(ref, seg):
        m0 = jnp.max(seg, axis=1, keepdims=True)
        sstar = jnp.min(jnp.where(seg == m0, iota, BIGI), axis=1,
                        keepdims=True)
        outs = []
        for r in range(RB):
            sv = sstar[r, 0]
            chunk = ref[r, pl.ds(sv, 1), :]
            eq = chunk.reshape(1, 1, SEG_W) == m0[r, 0]
            p = jnp.min(jnp.where(eq, iota200, BIGI))
            outs.append((sv * SEG_W + p).reshape(1, 1))
        return jnp.concatenate(outs, axis=0)

    amaxl = row_argmax(log_ref, segl)

    y = knn_ref[...]
    m0k = jnp.max(y, axis=1, keepdims=True)
    iota_v = jax.lax.broadcasted_iota(jnp.int32, (RB, V), 1)
    amaxk = jnp.min(jnp.where(y == m0k, iota_v, BIGI), axis=1, keepdims=True)
    lab = lab_ref[...]
    target_ref[...] = jnp.logical_and(amaxl != lab, amaxk == lab)

    h = jax.lax.dot_general(feat_ref[...], w1_ref[...],
                            (((1,), (1,)), ((), ())),
                            preferred_element_type=jnp.float32)
    h = h + b1_ref[...]
    h = jnp.where(h >= 0, h, 0.1 * h)
    featdot_ref[...] = (
        jnp.sum(h * w2f_ref[...], axis=1, keepdims=True) + b2_ref[0, 0])


def _k4_body(cand_ref, featdot_ref, w2t_ref, out_ref):
    c = cand_ref[...]
    chunk_iota = jax.lax.broadcasted_iota(jnp.int32, (RB, TOP_K, SEG_W), 1)
    pos_iota = chunk_iota * SEG_W + jax.lax.broadcasted_iota(
        jnp.int32, (RB, TOP_K, SEG_W), 2)
    acc = featdot_ref[...]
    for t in range(TOP_K):
        m = jnp.max(c, axis=(1, 2), keepdims=True)
        acc = acc + m[:, :, 0] * w2t_ref[0, t]
        pos = jnp.min(jnp.where(c == m, pos_iota, BIGI), axis=(1, 2),
                      keepdims=True)
        c = jnp.where(pos_iota == pos, NEG, c)
    out_ref[...] = acc


def kernel(normalized_features, logits, knn_logits, labels, W1, b1, W2, b2):
    nb = B // RB
    logits3 = logits.reshape(B, N_SEG, SEG_W)
    labels2 = labels.astype(jnp.int32).reshape(B, 1)
    w2t = W2[:, :TOP_K]
    w2f = W2[:, TOP_K:]

    cands, featdot, target = pl.pallas_call(
        _k1_body,
        grid=(nb,),
        in_specs=[
            pl.BlockSpec((RB, N_SEG, SEG_W), lambda i: (i, 0, 0)),
            pl.BlockSpec((RB, V), lambda i: (i, 0)),
            pl.BlockSpec((RB, 1), lambda i: (i, 0)),
            pl.BlockSpec((RB, FEAT_DIM), lambda i: (i, 0)),
            pl.BlockSpec((NORM_DIM, FEAT_DIM), lambda i: (0, 0)),
            pl.BlockSpec((1, NORM_DIM), lambda i: (0, 0)),
            pl.BlockSpec((1, NORM_DIM), lambda i: (0, 0)),
            pl.BlockSpec((1, 1), lambda i: (0, 0)),
        ],
        out_specs=[
            pl.BlockSpec((RB, TOP_K, SEG_W), lambda i: (i, 0, 0)),
            pl.BlockSpec((RB, 1), lambda i: (i, 0)),
            pl.BlockSpec((RB, 1), lambda i: (i, 0)),
        ],
        out_shape=[
            jax.ShapeDtypeStruct((B, TOP_K, SEG_W), jnp.float32),
            jax.ShapeDtypeStruct((B, 1), jnp.float32),
            jax.ShapeDtypeStruct((B, 1), jnp.bool_),
        ],
    )(logits3, knn_logits, labels2, normalized_features,
      W1, b1.reshape(1, NORM_DIM), w2f, b2.reshape(1, 1))

    out = pl.pallas_call(
        _k4_body,
        grid=(nb,),
        in_specs=[
            pl.BlockSpec((RB, TOP_K, SEG_W), lambda i: (i, 0, 0)),
            pl.BlockSpec((RB, 1), lambda i: (i, 0)),
            pl.BlockSpec((1, TOP_K), lambda i: (0, 0)),
        ],
        out_specs=pl.BlockSpec((RB, 1), lambda i: (i, 0)),
        out_shape=jax.ShapeDtypeStruct((B, 1), jnp.float32),
    )(cands, featdot, w2t)

    return out.reshape(B), target.reshape(B)


# speedup vs baseline: 2.0830x; 1.0001x over previous
import jax
import jax.numpy as jnp
from jax.experimental import pallas as pl

B = 1024
V = 100000
FEAT_DIM = 512
TOP_K = 30
NORM_DIM = 16

SEG_W = 125
N_SEG = V // SEG_W
RB = 8
NEG = float("-inf")
BIGI = 2 ** 30


def _k1_body(log_ref, knn_ref, lab_ref, feat_ref, w1_ref, b1_ref,
             w2f_ref, b2_ref, cand_ref, featdot_ref, target_ref):
    x = log_ref[...]
    segl = jnp.max(x, axis=2)

    iota = jax.lax.broadcasted_iota(jnp.int32, (RB, N_SEG), 1)
    s = segl
    for t in range(TOP_K):
        m = jnp.max(s, axis=1, keepdims=True)
        pos = jnp.min(jnp.where(s == m, iota, BIGI), axis=1, keepdims=True)
        s = jnp.where(iota == pos, NEG, s)
        for r in range(RB):
            sv = pos[r, 0]
            cand_ref[pl.ds(r, 1), pl.ds(t, 1), :] = (
                log_ref[pl.ds(r, 1), pl.ds(sv, 1), :])

    iota200 = jax.lax.broadcasted_iota(jnp.int32, (1, 1, SEG_W), 2)

    def row_argmax(ref, seg):
        m0 = jnp.max(seg, axis=1, keepdims=True)
        sstar = jnp.min(jnp.where(seg == m0, iota, BIGI), axis=1,
                        keepdims=True)
        outs = []
        for r in range(RB):
            sv = sstar[r, 0]
            chunk = ref[r, pl.ds(sv, 1), :]
            eq = chunk.reshape(1, 1, SEG_W) == m0[r, 0]
            p = jnp.min(jnp.where(eq, iota200, BIGI))
            outs.append((sv * SEG_W + p).reshape(1, 1))
        return jnp.concatenate(outs, axis=0)

    amaxl = row_argmax(log_ref, segl)

    y = knn_ref[...]
    m0k = jnp.max(y, axis=1, keepdims=True)
    iota_v = jax.lax.broadcasted_iota(jnp.int32, (RB, V), 1)
    amaxk = jnp.min(jnp.where(y == m0k, iota_v, BIGI), axis=1, keepdims=True)
    lab = lab_ref[...]
    target_ref[...] = jnp.logical_and(amaxl != lab, amaxk == lab)

    h = jax.lax.dot_general(feat_ref[...], w1_ref[...],
                            (((1,), (1,)), ((), ())),
                            preferred_element_type=jnp.float32)
    h = h + b1_ref[...]
    h = jnp.where(h >= 0, h, 0.1 * h)
    featdot_ref[...] = (
        jnp.sum(h * w2f_ref[...], axis=1, keepdims=True) + b2_ref[0, 0])


def _k4_body(cand_ref, featdot_ref, w2t_ref, out_ref):
    c = cand_ref[...]
    chunk_iota = jax.lax.broadcasted_iota(jnp.int32, (RB, TOP_K, SEG_W), 1)
    pos_iota = chunk_iota * SEG_W + jax.lax.broadcasted_iota(
        jnp.int32, (RB, TOP_K, SEG_W), 2)
    acc = featdot_ref[...]
    for t in range(TOP_K):
        m = jnp.max(c, axis=(1, 2), keepdims=True)
        acc = acc + m[:, :, 0] * w2t_ref[0, t]
        pos = jnp.min(jnp.where(c == m, pos_iota, BIGI), axis=(1, 2),
                      keepdims=True)
        c = jnp.where(pos_iota == pos, NEG, c)
    out_ref[...] = acc


def kernel(normalized_features, logits, knn_logits, labels, W1, b1, W2, b2):
    nb = B // RB
    logits3 = logits.reshape(B, N_SEG, SEG_W)
    labels2 = labels.astype(jnp.int32).reshape(B, 1)
    w2t = W2[:, :TOP_K]
    w2f = W2[:, TOP_K:]

    cands, featdot, target = pl.pallas_call(
        _k1_body,
        grid=(nb,),
        in_specs=[
            pl.BlockSpec((RB, N_SEG, SEG_W), lambda i: (i, 0, 0)),
            pl.BlockSpec((RB, V), lambda i: (i, 0)),
            pl.BlockSpec((RB, 1), lambda i: (i, 0)),
            pl.BlockSpec((RB, FEAT_DIM), lambda i: (i, 0)),
            pl.BlockSpec((NORM_DIM, FEAT_DIM), lambda i: (0, 0)),
            pl.BlockSpec((1, NORM_DIM), lambda i: (0, 0)),
            pl.BlockSpec((1, NORM_DIM), lambda i: (0, 0)),
            pl.BlockSpec((1, 1), lambda i: (0, 0)),
        ],
        out_specs=[
            pl.BlockSpec((RB, TOP_K, SEG_W), lambda i: (i, 0, 0)),
            pl.BlockSpec((RB, 1), lambda i: (i, 0)),
            pl.BlockSpec((RB, 1), lambda i: (i, 0)),
        ],
        out_shape=[
            jax.ShapeDtypeStruct((B, TOP_K, SEG_W), jnp.float32),
            jax.ShapeDtypeStruct((B, 1), jnp.float32),
            jax.ShapeDtypeStruct((B, 1), jnp.bool_),
        ],
    )(logits3, knn_logits, labels2, normalized_features,
      W1, b1.reshape(1, NORM_DIM), w2f, b2.reshape(1, 1))

    out = pl.pallas_call(
        _k4_body,
        grid=(nb,),
        in_specs=[
            pl.BlockSpec((RB, TOP_K, SEG_W), lambda i: (i, 0, 0)),
            pl.BlockSpec((RB, 1), lambda i: (i, 0)),
            pl.BlockSpec((1, TOP_K), lambda i: (0, 0)),
        ],
        out_specs=pl.BlockSpec((RB, 1), lambda i: (i, 0)),
        out_shape=jax.ShapeDtypeStruct((B, 1), jnp.float32),
    )(cands, featdot, w2t)

    return out.reshape(B), target.reshape(B)
